# baseline (speedup 1.0000x reference)
import jax
import jax.numpy as jnp
from jax import lax
from jax.experimental import pallas as pl
from jax.experimental.pallas import tpu as pltpu
from jax.experimental.pallas import tpu_sc as plsc

NUM_EDGES = 1000000
MEMORY_DIM = 64
BATCH = 16384

_info = plsc.get_sparse_core_info()
_NC, _NS = _info.num_cores, _info.num_subcores
_NW = _NC * _NS
_CW = 512
_GPW = 61
_SPAN = _GPW * _CW
_EXTRA_BASE = _NW * _SPAN
_REM_BASE = _EXTRA_BASE + _CW
_REM_W = NUM_EDGES - _REM_BASE
_CAPW = 2048
_CAPC = 128
_NB = BATCH + 16


def _scan_body(idx_hbm, memt_hbm, lu_hbm,
               vals_hbm, pos_hbm, luv_hbm,
               idx_v, me_v, mi_v, ce_v, ci_v,
               chunk_v, luch_v, stage_v, pos_acc, lu_acc,
               csem0, csem1, fsem):
    wid = lax.axis_index("s") * _NC + lax.axis_index("c")
    lo = wid * _SPAN
    hi = jnp.where(wid == _NW - 1, _REM_BASE, lo + _SPAN)

    pltpu.sync_copy(idx_hbm, idx_v)

    def f1(it, m):
        v = idx_v[pl.ds(it * 16, 16)]
        p = lax.iota(jnp.int32, 16) + it * 16
        mask = (v >= lo) & (v < hi)
        m_use = jnp.minimum(m, _CAPW - 16)
        plsc.store_compressed(me_v.at[pl.ds(m_use, 16)], v, mask=mask)
        plsc.store_compressed(mi_v.at[pl.ds(m_use, 16)], p, mask=mask)
        return m + plsc.all_reduce_population_count(mask)[0]

    m = lax.fori_loop(0, BATCH // 16, f1, jnp.int32(0))
    me_v[pl.ds(jnp.minimum(m, _CAPW - 16), 16)] = jnp.full((16,), -1, jnp.int32)

    def pf(i, _):
        pos_acc[pl.ds(i * 16, 16)] = jnp.full((16,), BATCH, jnp.int32)
        return _

    lax.fori_loop(0, _CAPW // 16, pf, jnp.int32(0))

    n_groups = (m + 15) // 16

    def process(base, width, parity, total_pad):
        def f2(it, m2):
            v = me_v[pl.ds(it * 16, 16)]
            p = mi_v[pl.ds(it * 16, 16)]
            mask = (v >= base) & (v < base + width)
            m2_use = jnp.minimum(m2, _CAPC - 16)
            plsc.store_compressed(ce_v.at[pl.ds(m2_use, 16)], v - base,
                                  mask=mask)
            plsc.store_compressed(ci_v.at[pl.ds(m2_use, 16)], p, mask=mask)
            return m2 + plsc.all_reduce_population_count(mask)[0]

        m2 = lax.fori_loop(0, n_groups, f2, jnp.int32(0))
        m2 = jnp.minimum(m2, _CAPC - 16)
        ce_v[pl.ds(m2, 16)] = jnp.zeros((16,), jnp.int32)
        ci_v[pl.ds(m2, 16)] = jnp.full((16,), BATCH, jnp.int32)
        n_g2 = (m2 + 15) // 16
        pv = jnp.full((16,), parity, jnp.int32)
        lubase = parity * _CW

        def extract(g2, _):
            cols16 = ce_v[pl.ds(g2 * 16, 16)]
            pos16 = ci_v[pl.ds(g2 * 16, 16)]
            lu16 = plsc.load_gather(luch_v, [lubase + cols16])
            off = jnp.minimum(total_pad + g2 * 16, _CAPW - 16)
            pos_acc[pl.ds(off, 16)] = pos16
            lu_acc[pl.ds(off, 16)] = lu16
            sbase = (parity * _CAPC + g2 * 16) * MEMORY_DIM
            for k in range(16):
                colv = jnp.full((16,), cols16[k], jnp.int32)
                for j in range(MEMORY_DIM // 16):
                    rows = lax.iota(jnp.int32, 16) + j * 16
                    vals = plsc.load_gather(chunk_v, [pv, rows, colv])
                    stage_v[pl.ds(sbase + k * MEMORY_DIM + j * 16, 16)] = vals
            return _

        lax.fori_loop(0, n_g2, extract, jnp.int32(0))

        def flush(u, _):
            row = jnp.minimum(total_pad + u * 16, _CAPW - 16)
            doff = pl.multiple_of((wid * _CAPW + row) * MEMORY_DIM, 1024)
            soff = pl.multiple_of(
                (parity * _CAPC + u * 16) * MEMORY_DIM, 1024)
            pltpu.async_copy(stage_v.at[pl.ds(soff, 16 * MEMORY_DIM)],
                             vals_hbm.at[pl.ds(doff, 16 * MEMORY_DIM)], fsem)
            return _

        lax.fori_loop(0, n_g2, flush, jnp.int32(0))
        return total_pad + n_g2 * 16, n_g2

    chunk_like = memt_hbm.at[:, pl.ds(0, _CW)]
    lu_like = lu_hbm.at[pl.ds(0, _CW)]

    def start_load(g, parity_slot, sem):
        base = pl.multiple_of(lo + g * _CW, _CW)
        pltpu.async_copy(memt_hbm.at[:, pl.ds(base, _CW)],
                         chunk_v.at[parity_slot], sem)
        pltpu.async_copy(lu_hbm.at[pl.ds(base, _CW)],
                         luch_v.at[pl.ds(parity_slot * _CW, _CW)], sem)

    start_load(jnp.int32(0), 0, csem0)

    def drain_unit(i, _):
        pltpu.make_async_copy(
            vals_hbm.at[pl.ds(0, 16 * MEMORY_DIM)],
            stage_v.at[pl.ds(0, 16 * MEMORY_DIM)], fsem).wait()
        return _

    def body(g, carry):
        total_pad, u0, u1 = carry
        parity = g % 2

        @pl.when(parity == 0)
        def _():
            pltpu.make_async_copy(chunk_like, chunk_v.at[0], csem0).wait()
            pltpu.make_async_copy(
                lu_like, luch_v.at[pl.ds(0, _CW)], csem0).wait()

        @pl.when(parity == 1)
        def _():
            pltpu.make_async_copy(chunk_like, chunk_v.at[1], csem1).wait()
            pltpu.make_async_copy(
                lu_like, luch_v.at[pl.ds(_CW, _CW)], csem1).wait()

        @pl.when((g + 1 < _GPW) & (parity == 0))
        def _():
            start_load(g + 1, 1, csem1)

        @pl.when((g + 1 < _GPW) & (parity == 1))
        def _():
            start_load(g + 1, 0, csem0)

        u_prev = jnp.where(parity == 0, u0, u1)
        lax.fori_loop(0, u_prev, drain_unit, jnp.int32(0))

        base = lo + g * _CW
        total_pad, n_u = process(base, _CW, parity, total_pad)
        u0 = jnp.where(parity == 0, n_u, u0)
        u1 = jnp.where(parity == 1, n_u, u1)
        return total_pad, u0, u1

    total_pad, u0, u1 = lax.fori_loop(
        0, _GPW, body, (jnp.int32(0), jnp.int32(0), jnp.int32(0)))

    lax.fori_loop(0, u0 + u1, drain_unit, jnp.int32(0))

    @pl.when(wid == _NW - 1)
    def _():
        pltpu.sync_copy(memt_hbm.at[:, pl.ds(_EXTRA_BASE, _CW)],
                        chunk_v.at[0])
        pltpu.sync_copy(lu_hbm.at[pl.ds(_EXTRA_BASE, _CW)],
                        luch_v.at[pl.ds(0, _CW)])
        tp2, nu2 = process(jnp.int32(_EXTRA_BASE), _CW, 0, total_pad)
        lax.fori_loop(0, nu2, drain_unit, jnp.int32(0))

    poff = pl.multiple_of(wid * _CAPW, _CAPW)
    pltpu.sync_copy(pos_acc, pos_hbm.at[pl.ds(poff, _CAPW)])
    pltpu.sync_copy(lu_acc, luv_hbm.at[pl.ds(poff, _CAPW)])


def _scatter_body(vals_hbm, pos3_hbm, luv_hbm, out_hbm, luo_hbm,
                  pidx_v, luv_v, vstage_v, ssem0, ssem1, wsem):
    wid = lax.axis_index("s") * _NC + lax.axis_index("c")
    pltpu.sync_copy(pos3_hbm.at[wid], pidx_v)
    pltpu.sync_copy(luv_hbm.at[wid], luv_v)
    nj = _CAPW // 128
    scat_d = [None] * nj
    for j in range(nj):
        s = j % 2
        sem = ssem0 if s == 0 else ssem1
        if j >= 2:
            scat_d[j - 2][0].wait()
            scat_d[j - 2][1].wait()
        pltpu.async_copy(
            vals_hbm.at[pl.ds(wid * _CAPW + j * 128, 128)],
            vstage_v.at[s], sem).wait()
        scat_d[j] = (
            pltpu.async_copy(vstage_v.at[s], out_hbm.at[pidx_v.at[j]], wsem),
            pltpu.async_copy(luv_v.at[pl.ds(j * 128, 128)],
                             luo_hbm.at[pidx_v.at[j]], wsem),
        )
    for j in range(nj - 2, nj):
        scat_d[j][0].wait()
        scat_d[j][1].wait()


@jax.jit
def _edge_gather(e_id32, memt, last_update):
    mesh = plsc.VectorSubcoreMesh(core_axis_name="c", subcore_axis_name="s")
    vals, pos, luv = pl.kernel(
        _scan_body,
        mesh=mesh,
        out_type=(
            jax.ShapeDtypeStruct((_NW * _CAPW * MEMORY_DIM,), jnp.float32),
            jax.ShapeDtypeStruct((_NW * _CAPW,), jnp.int32),
            jax.ShapeDtypeStruct((_NW * _CAPW,), jnp.int32),
        ),
        scratch_types=[
            pltpu.VMEM((BATCH,), jnp.int32),
            pltpu.VMEM((_CAPW,), jnp.int32),
            pltpu.VMEM((_CAPW,), jnp.int32),
            pltpu.VMEM((_CAPC,), jnp.int32),
            pltpu.VMEM((_CAPC,), jnp.int32),
            pltpu.VMEM((2, MEMORY_DIM, _CW), jnp.float32),
            pltpu.VMEM((2 * _CW,), jnp.int32),
            pltpu.VMEM((2 * _CAPC * MEMORY_DIM,), jnp.float32),
            pltpu.VMEM((_CAPW,), jnp.int32),
            pltpu.VMEM((_CAPW,), jnp.int32),
            pltpu.SemaphoreType.DMA,
            pltpu.SemaphoreType.DMA,
            pltpu.SemaphoreType.DMA,
        ],
        compiler_params=pltpu.CompilerParams(needs_layout_passes=False),
    )(e_id32, memt, last_update)

    vals2d = vals.reshape(_NW * _CAPW, MEMORY_DIM)
    pos3 = pos.reshape(_NW, _CAPW // 128, 128)
    luv2 = luv.reshape(_NW, _CAPW)
    out_pad, luo_pad = pl.kernel(
        _scatter_body,
        mesh=mesh,
        out_type=(
            jax.ShapeDtypeStruct((_NB, MEMORY_DIM), jnp.float32),
            jax.ShapeDtypeStruct((_NB,), jnp.int32),
        ),
        scratch_types=[
            pltpu.VMEM((_CAPW // 128, 128), jnp.int32),
            pltpu.VMEM((_CAPW,), jnp.int32),
            pltpu.VMEM((2, 128, MEMORY_DIM), jnp.float32),
            pltpu.SemaphoreType.DMA,
            pltpu.SemaphoreType.DMA,
            pltpu.SemaphoreType.DMA,
        ],
        compiler_params=pltpu.CompilerParams(
            needs_layout_passes=False, use_tc_tiling_on_sc=False),
    )(vals2d, pos3, luv2)
    return out_pad[:BATCH], luo_pad[:BATCH]


def kernel(e_id, memory, last_update):
    e32 = e_id.astype(jnp.int32)
    mem_out, lu_out = _edge_gather(e32, memory.T, last_update)
    in_rem = e32 >= _REM_BASE
    e_rel = jnp.clip(e32 - _REM_BASE, 0, _REM_W - 1)
    mem_rem = jnp.take(memory[_REM_BASE:], e_rel, axis=0)
    lu_rem = jnp.take(last_update[_REM_BASE:], e_rel, axis=0)
    mem_out = jnp.where(in_rem[:, None], mem_rem, mem_out)
    lu_out = jnp.where(in_rem, lu_rem, lu_out)
    return (mem_out, lu_out.astype(last_update.dtype))


# speedup vs baseline: 1.0007x; 1.0007x over previous
import jax
import jax.numpy as jnp
from jax import lax
from jax.experimental import pallas as pl
from jax.experimental.pallas import tpu as pltpu
from jax.experimental.pallas import tpu_sc as plsc

NUM_EDGES = 1000000
MEMORY_DIM = 64
BATCH = 16384

_info = plsc.get_sparse_core_info()
_NC, _NS = _info.num_cores, _info.num_subcores
_NW = _NC * _NS
_CW = 512
_GPW = 61
_SPAN = _GPW * _CW
_EXTRA_BASE = _NW * _SPAN
_REM_BASE = _EXTRA_BASE + _CW
_REM_W = NUM_EDGES - _REM_BASE
_CAPW = 2048
_CAPC = 128
_NB = BATCH + 16


def _scan_body(idx_hbm, memt_hbm, lu_hbm,
               vals_hbm, pos_hbm, luv_hbm, cnt_hbm,
               idx_v, me_v, mi_v, ce_v, ci_v,
               chunk_v, luch_v, stage_v, pos_acc, lu_acc,
               csem0, csem1, fsem):
    wid = lax.axis_index("s") * _NC + lax.axis_index("c")
    lo = wid * _SPAN
    hi = jnp.where(wid == _NW - 1, _REM_BASE, lo + _SPAN)

    pltpu.sync_copy(idx_hbm, idx_v)

    def f1(it, m):
        v = idx_v[pl.ds(it * 16, 16)]
        p = lax.iota(jnp.int32, 16) + it * 16
        mask = (v >= lo) & (v < hi)
        m_use = jnp.minimum(m, _CAPW - 16)
        plsc.store_compressed(me_v.at[pl.ds(m_use, 16)], v, mask=mask)
        plsc.store_compressed(mi_v.at[pl.ds(m_use, 16)], p, mask=mask)
        return m + plsc.all_reduce_population_count(mask)[0]

    m = lax.fori_loop(0, BATCH // 16, f1, jnp.int32(0))
    me_v[pl.ds(jnp.minimum(m, _CAPW - 16), 16)] = jnp.full((16,), -1, jnp.int32)

    def pf(i, _):
        pos_acc[pl.ds(i * 16, 16)] = lax.iota(jnp.int32, 16) + BATCH
        return _

    lax.fori_loop(0, _CAPW // 16, pf, jnp.int32(0))

    n_groups = (m + 15) // 16

    def process(base, width, parity, total_pad):
        def f2(it, m2):
            v = me_v[pl.ds(it * 16, 16)]
            p = mi_v[pl.ds(it * 16, 16)]
            mask = (v >= base) & (v < base + width)
            m2_use = jnp.minimum(m2, _CAPC - 16)
            plsc.store_compressed(ce_v.at[pl.ds(m2_use, 16)], v - base,
                                  mask=mask)
            plsc.store_compressed(ci_v.at[pl.ds(m2_use, 16)], p, mask=mask)
            return m2 + plsc.all_reduce_population_count(mask)[0]

        m2 = lax.fori_loop(0, n_groups, f2, jnp.int32(0))
        m2 = jnp.minimum(m2, _CAPC - 16)
        ce_v[pl.ds(m2, 16)] = jnp.zeros((16,), jnp.int32)
        ci_v[pl.ds(m2, 16)] = lax.iota(jnp.int32, 16) + BATCH
        n_g2 = (m2 + 15) // 16
        pv = jnp.full((16,), parity, jnp.int32)
        lubase = parity * _CW

        def extract(g2, _):
            cols16 = ce_v[pl.ds(g2 * 16, 16)]
            pos16 = ci_v[pl.ds(g2 * 16, 16)]
            lu16 = plsc.load_gather(luch_v, [lubase + cols16])
            off = jnp.minimum(total_pad + g2 * 16, _CAPW - 16)
            pos_acc[pl.ds(off, 16)] = pos16
            lu_acc[pl.ds(off, 16)] = lu16
            sbase = (parity * _CAPC + g2 * 16) * MEMORY_DIM
            for k in range(16):
                colv = jnp.full((16,), cols16[k], jnp.int32)
                for j in range(MEMORY_DIM // 16):
                    rows = lax.iota(jnp.int32, 16) + j * 16
                    vals = plsc.load_gather(chunk_v, [pv, rows, colv])
                    stage_v[pl.ds(sbase + k * MEMORY_DIM + j * 16, 16)] = vals
            return _

        lax.fori_loop(0, n_g2, extract, jnp.int32(0))

        def flush(u, _):
            row = jnp.minimum(total_pad + u * 16, _CAPW - 16)
            doff = pl.multiple_of((wid * _CAPW + row) * MEMORY_DIM, 1024)
            soff = pl.multiple_of(
                (parity * _CAPC + u * 16) * MEMORY_DIM, 1024)
            pltpu.async_copy(stage_v.at[pl.ds(soff, 16 * MEMORY_DIM)],
                             vals_hbm.at[pl.ds(doff, 16 * MEMORY_DIM)], fsem)
            return _

        lax.fori_loop(0, n_g2, flush, jnp.int32(0))
        return total_pad + n_g2 * 16, n_g2

    chunk_like = memt_hbm.at[:, pl.ds(0, _CW)]
    lu_like = lu_hbm.at[pl.ds(0, _CW)]

    def start_load(g, parity_slot, sem):
        base = pl.multiple_of(lo + g * _CW, _CW)
        pltpu.async_copy(memt_hbm.at[:, pl.ds(base, _CW)],
                         chunk_v.at[parity_slot], sem)
        pltpu.async_copy(lu_hbm.at[pl.ds(base, _CW)],
                         luch_v.at[pl.ds(parity_slot * _CW, _CW)], sem)

    start_load(jnp.int32(0), 0, csem0)

    def drain_unit(i, _):
        pltpu.make_async_copy(
            vals_hbm.at[pl.ds(0, 16 * MEMORY_DIM)],
            stage_v.at[pl.ds(0, 16 * MEMORY_DIM)], fsem).wait()
        return _

    def body(g, carry):
        total_pad, u0, u1 = carry
        parity = g % 2

        @pl.when(parity == 0)
        def _():
            pltpu.make_async_copy(chunk_like, chunk_v.at[0], csem0).wait()
            pltpu.make_async_copy(
                lu_like, luch_v.at[pl.ds(0, _CW)], csem0).wait()

        @pl.when(parity == 1)
        def _():
            pltpu.make_async_copy(chunk_like, chunk_v.at[1], csem1).wait()
            pltpu.make_async_copy(
                lu_like, luch_v.at[pl.ds(_CW, _CW)], csem1).wait()

        @pl.when((g + 1 < _GPW) & (parity == 0))
        def _():
            start_load(g + 1, 1, csem1)

        @pl.when((g + 1 < _GPW) & (parity == 1))
        def _():
            start_load(g + 1, 0, csem0)

        u_prev = jnp.where(parity == 0, u0, u1)
        lax.fori_loop(0, u_prev, drain_unit, jnp.int32(0))

        base = lo + g * _CW
        total_pad, n_u = process(base, _CW, parity, total_pad)
        u0 = jnp.where(parity == 0, n_u, u0)
        u1 = jnp.where(parity == 1, n_u, u1)
        return total_pad, u0, u1

    total_pad, u0, u1 = lax.fori_loop(
        0, _GPW, body, (jnp.int32(0), jnp.int32(0), jnp.int32(0)))

    lax.fori_loop(0, u0 + u1, drain_unit, jnp.int32(0))

    def write_count(tp):
        ce_v[pl.ds(0, 16)] = jnp.full((16,), tp, jnp.int32)
        coff = pl.multiple_of(wid * 16, 16)
        pltpu.sync_copy(ce_v.at[pl.ds(0, 16)], cnt_hbm.at[pl.ds(coff, 16)])

    @pl.when(wid == _NW - 1)
    def _():
        pltpu.sync_copy(memt_hbm.at[:, pl.ds(_EXTRA_BASE, _CW)],
                        chunk_v.at[0])
        pltpu.sync_copy(lu_hbm.at[pl.ds(_EXTRA_BASE, _CW)],
                        luch_v.at[pl.ds(0, _CW)])
        tp2, nu2 = process(jnp.int32(_EXTRA_BASE), _CW, 0, total_pad)
        lax.fori_loop(0, nu2, drain_unit, jnp.int32(0))
        write_count(tp2)

    @pl.when(wid != _NW - 1)
    def _():
        write_count(total_pad)

    poff = pl.multiple_of(wid * _CAPW, _CAPW)
    pltpu.sync_copy(pos_acc, pos_hbm.at[pl.ds(poff, _CAPW)])
    pltpu.sync_copy(lu_acc, luv_hbm.at[pl.ds(poff, _CAPW)])


def _scatter_body(vals_hbm, pos3_hbm, luv_hbm, cnt_hbm, out_hbm, luo_hbm,
                  pidx_v, luv_v, cnt_v, vstage_v, ssem0, ssem1, wsem):
    wid = lax.axis_index("s") * _NC + lax.axis_index("c")
    pltpu.sync_copy(pos3_hbm.at[wid], pidx_v)
    pltpu.sync_copy(luv_hbm.at[wid], luv_v)
    pltpu.sync_copy(cnt_hbm.at[wid], cnt_v)
    cnt = cnt_v[pl.ds(0, 16)][0]
    nj = _CAPW // 128
    scat_d = [None] * nj

    def chunk(j):
        s = j % 2
        sem = ssem0 if s == 0 else ssem1
        pltpu.async_copy(
            vals_hbm.at[pl.ds(wid * _CAPW + j * 128, 128)],
            vstage_v.at[s], sem).wait()
        scat_d[j] = (
            pltpu.async_copy(vstage_v.at[s],
                             out_hbm.at[pidx_v.at[j]], wsem),
            pltpu.async_copy(luv_v.at[pl.ds(j * 128, 128)],
                             luo_hbm.at[pidx_v.at[j]], wsem),
        )

    def wait_chunk(j):
        scat_d[j][0].wait()
        scat_d[j][1].wait()

    for j in range(nj):
        if j >= 2:
            wait_chunk(j - 2)
        chunk(j)
    for j in range(nj - 2, nj):
        wait_chunk(j)


@jax.jit
def _edge_gather(e_id32, memt, last_update):
    mesh = plsc.VectorSubcoreMesh(core_axis_name="c", subcore_axis_name="s")
    vals, pos, luv, cnt = pl.kernel(
        _scan_body,
        mesh=mesh,
        out_type=(
            jax.ShapeDtypeStruct((_NW * _CAPW * MEMORY_DIM,), jnp.float32),
            jax.ShapeDtypeStruct((_NW * _CAPW,), jnp.int32),
            jax.ShapeDtypeStruct((_NW * _CAPW,), jnp.int32),
            jax.ShapeDtypeStruct((_NW * 16,), jnp.int32),
        ),
        scratch_types=[
            pltpu.VMEM((BATCH,), jnp.int32),
            pltpu.VMEM((_CAPW,), jnp.int32),
            pltpu.VMEM((_CAPW,), jnp.int32),
            pltpu.VMEM((_CAPC,), jnp.int32),
            pltpu.VMEM((_CAPC,), jnp.int32),
            pltpu.VMEM((2, MEMORY_DIM, _CW), jnp.float32),
            pltpu.VMEM((2 * _CW,), jnp.int32),
            pltpu.VMEM((2 * _CAPC * MEMORY_DIM,), jnp.float32),
            pltpu.VMEM((_CAPW,), jnp.int32),
            pltpu.VMEM((_CAPW,), jnp.int32),
            pltpu.SemaphoreType.DMA,
            pltpu.SemaphoreType.DMA,
            pltpu.SemaphoreType.DMA,
        ],
        compiler_params=pltpu.CompilerParams(needs_layout_passes=False),
    )(e_id32, memt, last_update)

    vals2d = vals.reshape(_NW * _CAPW, MEMORY_DIM)
    pos3 = pos.reshape(_NW, _CAPW // 128, 128)
    luv2 = luv.reshape(_NW, _CAPW)
    cnt2 = cnt.reshape(_NW, 16)
    out_pad, luo_pad = pl.kernel(
        _scatter_body,
        mesh=mesh,
        out_type=(
            jax.ShapeDtypeStruct((_NB, MEMORY_DIM), jnp.float32),
            jax.ShapeDtypeStruct((_NB,), jnp.int32),
        ),
        scratch_types=[
            pltpu.VMEM((_CAPW // 128, 128), jnp.int32),
            pltpu.VMEM((_CAPW,), jnp.int32),
            pltpu.VMEM((16,), jnp.int32),
            pltpu.VMEM((2, 128, MEMORY_DIM), jnp.float32),
            pltpu.SemaphoreType.DMA,
            pltpu.SemaphoreType.DMA,
            pltpu.SemaphoreType.DMA,
        ],
        compiler_params=pltpu.CompilerParams(
            needs_layout_passes=False, use_tc_tiling_on_sc=False),
    )(vals2d, pos3, luv2, cnt2)
    return out_pad[:BATCH], luo_pad[:BATCH]


def kernel(e_id, memory, last_update):
    e32 = e_id.astype(jnp.int32)
    mem_out, lu_out = _edge_gather(e32, memory.T, last_update)
    in_rem = e32 >= _REM_BASE
    e_rel = jnp.clip(e32 - _REM_BASE, 0, _REM_W - 1)
    mem_rem = jnp.take(memory[_REM_BASE:], e_rel, axis=0)
    lu_rem = jnp.take(last_update[_REM_BASE:], e_rel, axis=0)
    mem_out = jnp.where(in_rem[:, None], mem_rem, mem_out)
    lu_out = jnp.where(in_rem, lu_rem, lu_out)
    return (mem_out, lu_out.astype(last_update.dtype))


# speedup vs baseline: 16.7519x; 16.7406x over previous
import jax
import jax.numpy as jnp
from jax import lax
from jax.experimental import pallas as pl
from jax.experimental.pallas import tpu as pltpu
from jax.experimental.pallas import tpu_sc as plsc

NUM_EDGES = 1000000
MEMORY_DIM = 64
BATCH = 16384

_info = plsc.get_sparse_core_info()
_NC, _NS = _info.num_cores, _info.num_subcores
_NW = _NC * _NS
_CW = 512
_GPW = 61
_SPAN = _GPW * _CW
_EXTRA_BASE = _NW * _SPAN
_REM_BASE = _EXTRA_BASE + _CW
_REM_W = NUM_EDGES - _REM_BASE
_CAPW = 2048
_CAPC = 128
_NB = BATCH + _NW * _CAPW


def _scan_body(idx_hbm, memt_hbm, lu_hbm,
               vals_hbm, pos_hbm, luv_hbm, cnt_hbm,
               idx_v, me_v, mi_v, ce_v, ci_v,
               chunk_v, luch_v, stage_v, pos_acc, lu_acc,
               csem0, csem1, fsem):
    wid = lax.axis_index("s") * _NC + lax.axis_index("c")
    lo = wid * _SPAN
    hi = jnp.where(wid == _NW - 1, _REM_BASE, lo + _SPAN)

    pltpu.sync_copy(idx_hbm, idx_v)

    def f1(it, m):
        v = idx_v[pl.ds(it * 16, 16)]
        p = lax.iota(jnp.int32, 16) + it * 16
        mask = (v >= lo) & (v < hi)
        m_use = jnp.minimum(m, _CAPW - 16)
        plsc.store_compressed(me_v.at[pl.ds(m_use, 16)], v, mask=mask)
        plsc.store_compressed(mi_v.at[pl.ds(m_use, 16)], p, mask=mask)
        return m + plsc.all_reduce_population_count(mask)[0]

    m = lax.fori_loop(0, BATCH // 16, f1, jnp.int32(0))
    me_v[pl.ds(jnp.minimum(m, _CAPW - 16), 16)] = jnp.full((16,), -1, jnp.int32)

    dummy0 = BATCH + wid * _CAPW

    def pf(i, _):
        pos_acc[pl.ds(i * 16, 16)] = lax.iota(jnp.int32, 16) + (
            dummy0 + i * 16)
        return _

    lax.fori_loop(0, _CAPW // 16, pf, jnp.int32(0))

    n_groups = (m + 15) // 16

    def process(base, width, parity, total_pad):
        def f2(it, m2):
            v = me_v[pl.ds(it * 16, 16)]
            p = mi_v[pl.ds(it * 16, 16)]
            mask = (v >= base) & (v < base + width)
            m2_use = jnp.minimum(m2, _CAPC - 16)
            plsc.store_compressed(ce_v.at[pl.ds(m2_use, 16)], v - base,
                                  mask=mask)
            plsc.store_compressed(ci_v.at[pl.ds(m2_use, 16)], p, mask=mask)
            return m2 + plsc.all_reduce_population_count(mask)[0]

        m2 = lax.fori_loop(0, n_groups, f2, jnp.int32(0))
        m2 = jnp.minimum(m2, _CAPC - 16)
        ce_v[pl.ds(m2, 16)] = jnp.zeros((16,), jnp.int32)
        ci_v[pl.ds(m2, 16)] = lax.iota(jnp.int32, 16) + (
            dummy0 + total_pad + m2)
        n_g2 = (m2 + 15) // 16
        pv = jnp.full((16,), parity, jnp.int32)
        lubase = parity * _CW

        def extract(g2, _):
            cols16 = ce_v[pl.ds(g2 * 16, 16)]
            pos16 = ci_v[pl.ds(g2 * 16, 16)]
            lu16 = plsc.load_gather(luch_v, [lubase + cols16])
            off = jnp.minimum(total_pad + g2 * 16, _CAPW - 16)
            pos_acc[pl.ds(off, 16)] = pos16
            lu_acc[pl.ds(off, 16)] = lu16
            sbase = (parity * _CAPC + g2 * 16) * MEMORY_DIM
            for k in range(16):
                colv = jnp.full((16,), cols16[k], jnp.int32)
                for j in range(MEMORY_DIM // 16):
                    rows = lax.iota(jnp.int32, 16) + j * 16
                    vals = plsc.load_gather(chunk_v, [pv, rows, colv])
                    stage_v[pl.ds(sbase + k * MEMORY_DIM + j * 16, 16)] = vals
            return _

        lax.fori_loop(0, n_g2, extract, jnp.int32(0))

        def flush(u, _):
            row = jnp.minimum(total_pad + u * 16, _CAPW - 16)
            doff = pl.multiple_of((wid * _CAPW + row) * MEMORY_DIM, 1024)
            soff = pl.multiple_of(
                (parity * _CAPC + u * 16) * MEMORY_DIM, 1024)
            pltpu.async_copy(stage_v.at[pl.ds(soff, 16 * MEMORY_DIM)],
                             vals_hbm.at[pl.ds(doff, 16 * MEMORY_DIM)], fsem)
            return _

        lax.fori_loop(0, n_g2, flush, jnp.int32(0))
        return total_pad + n_g2 * 16, n_g2

    chunk_like = memt_hbm.at[:, pl.ds(0, _CW)]
    lu_like = lu_hbm.at[pl.ds(0, _CW)]

    def start_load(g, parity_slot, sem):
        base = pl.multiple_of(lo + g * _CW, _CW)
        pltpu.async_copy(memt_hbm.at[:, pl.ds(base, _CW)],
                         chunk_v.at[parity_slot], sem)
        pltpu.async_copy(lu_hbm.at[pl.ds(base, _CW)],
                         luch_v.at[pl.ds(parity_slot * _CW, _CW)], sem)

    start_load(jnp.int32(0), 0, csem0)

    def drain_unit(i, _):
        pltpu.make_async_copy(
            vals_hbm.at[pl.ds(0, 16 * MEMORY_DIM)],
            stage_v.at[pl.ds(0, 16 * MEMORY_DIM)], fsem).wait()
        return _

    def body(g, carry):
        total_pad, u0, u1 = carry
        parity = g % 2

        @pl.when(parity == 0)
        def _():
            pltpu.make_async_copy(chunk_like, chunk_v.at[0], csem0).wait()
            pltpu.make_async_copy(
                lu_like, luch_v.at[pl.ds(0, _CW)], csem0).wait()

        @pl.when(parity == 1)
        def _():
            pltpu.make_async_copy(chunk_like, chunk_v.at[1], csem1).wait()
            pltpu.make_async_copy(
                lu_like, luch_v.at[pl.ds(_CW, _CW)], csem1).wait()

        @pl.when((g + 1 < _GPW) & (parity == 0))
        def _():
            start_load(g + 1, 1, csem1)

        @pl.when((g + 1 < _GPW) & (parity == 1))
        def _():
            start_load(g + 1, 0, csem0)

        u_prev = jnp.where(parity == 0, u0, u1)
        lax.fori_loop(0, u_prev, drain_unit, jnp.int32(0))

        base = lo + g * _CW
        total_pad, n_u = process(base, _CW, parity, total_pad)
        u0 = jnp.where(parity == 0, n_u, u0)
        u1 = jnp.where(parity == 1, n_u, u1)
        return total_pad, u0, u1

    total_pad, u0, u1 = lax.fori_loop(
        0, _GPW, body, (jnp.int32(0), jnp.int32(0), jnp.int32(0)))

    lax.fori_loop(0, u0 + u1, drain_unit, jnp.int32(0))

    def write_count(tp):
        ce_v[pl.ds(0, 16)] = jnp.full((16,), tp, jnp.int32)
        coff = pl.multiple_of(wid * 16, 16)
        pltpu.sync_copy(ce_v.at[pl.ds(0, 16)], cnt_hbm.at[pl.ds(coff, 16)])

    @pl.when(wid == _NW - 1)
    def _():
        pltpu.sync_copy(memt_hbm.at[:, pl.ds(_EXTRA_BASE, _CW)],
                        chunk_v.at[0])
        pltpu.sync_copy(lu_hbm.at[pl.ds(_EXTRA_BASE, _CW)],
                        luch_v.at[pl.ds(0, _CW)])
        tp2, nu2 = process(jnp.int32(_EXTRA_BASE), _CW, 0, total_pad)
        lax.fori_loop(0, nu2, drain_unit, jnp.int32(0))
        write_count(tp2)

    @pl.when(wid != _NW - 1)
    def _():
        write_count(total_pad)

    poff = pl.multiple_of(wid * _CAPW, _CAPW)
    pltpu.sync_copy(pos_acc, pos_hbm.at[pl.ds(poff, _CAPW)])
    pltpu.sync_copy(lu_acc, luv_hbm.at[pl.ds(poff, _CAPW)])


def _scatter_body(vals_hbm, pos3_hbm, luv_hbm, cnt_hbm, out_hbm, luo_hbm,
                  pidx_v, luv_v, cnt_v, vstage_v, ssem0, ssem1, wsem):
    wid = lax.axis_index("s") * _NC + lax.axis_index("c")
    pltpu.sync_copy(pos3_hbm.at[wid], pidx_v)
    pltpu.sync_copy(luv_hbm.at[wid], luv_v)
    pltpu.sync_copy(cnt_hbm.at[wid], cnt_v)
    cnt = cnt_v[pl.ds(0, 16)][0]
    nj = _CAPW // 128
    scat_d = [None] * nj

    def chunk(j):
        s = j % 2
        sem = ssem0 if s == 0 else ssem1
        pltpu.async_copy(
            vals_hbm.at[pl.ds(wid * _CAPW + j * 128, 128)],
            vstage_v.at[s], sem).wait()
        scat_d[j] = (
            pltpu.async_copy(vstage_v.at[s],
                             out_hbm.at[pidx_v.at[j]], wsem),
            pltpu.async_copy(luv_v.at[pl.ds(j * 128, 128)],
                             luo_hbm.at[pidx_v.at[j]], wsem),
        )

    def wait_chunk(j):
        scat_d[j][0].wait()
        scat_d[j][1].wait()

    for j in range(nj):
        if j >= 2:
            wait_chunk(j - 2)
        chunk(j)
    for j in range(nj - 2, nj):
        wait_chunk(j)


@jax.jit
def _edge_gather(e_id32, memt, last_update):
    mesh = plsc.VectorSubcoreMesh(core_axis_name="c", subcore_axis_name="s")
    vals, pos, luv, cnt = pl.kernel(
        _scan_body,
        mesh=mesh,
        out_type=(
            jax.ShapeDtypeStruct((_NW * _CAPW * MEMORY_DIM,), jnp.float32),
            jax.ShapeDtypeStruct((_NW * _CAPW,), jnp.int32),
            jax.ShapeDtypeStruct((_NW * _CAPW,), jnp.int32),
            jax.ShapeDtypeStruct((_NW * 16,), jnp.int32),
        ),
        scratch_types=[
            pltpu.VMEM((BATCH,), jnp.int32),
            pltpu.VMEM((_CAPW,), jnp.int32),
            pltpu.VMEM((_CAPW,), jnp.int32),
            pltpu.VMEM((_CAPC,), jnp.int32),
            pltpu.VMEM((_CAPC,), jnp.int32),
            pltpu.VMEM((2, MEMORY_DIM, _CW), jnp.float32),
            pltpu.VMEM((2 * _CW,), jnp.int32),
            pltpu.VMEM((2 * _CAPC * MEMORY_DIM,), jnp.float32),
            pltpu.VMEM((_CAPW,), jnp.int32),
            pltpu.VMEM((_CAPW,), jnp.int32),
            pltpu.SemaphoreType.DMA,
            pltpu.SemaphoreType.DMA,
            pltpu.SemaphoreType.DMA,
        ],
        compiler_params=pltpu.CompilerParams(needs_layout_passes=False),
    )(e_id32, memt, last_update)

    vals2d = vals.reshape(_NW * _CAPW, MEMORY_DIM)
    pos3 = pos.reshape(_NW, _CAPW // 128, 128)
    luv2 = luv.reshape(_NW, _CAPW)
    cnt2 = cnt.reshape(_NW, 16)
    out_pad, luo_pad = pl.kernel(
        _scatter_body,
        mesh=mesh,
        out_type=(
            jax.ShapeDtypeStruct((_NB, MEMORY_DIM), jnp.float32),
            jax.ShapeDtypeStruct((_NB,), jnp.int32),
        ),
        scratch_types=[
            pltpu.VMEM((_CAPW // 128, 128), jnp.int32),
            pltpu.VMEM((_CAPW,), jnp.int32),
            pltpu.VMEM((16,), jnp.int32),
            pltpu.VMEM((2, 128, MEMORY_DIM), jnp.float32),
            pltpu.SemaphoreType.DMA,
            pltpu.SemaphoreType.DMA,
            pltpu.SemaphoreType.DMA,
        ],
        compiler_params=pltpu.CompilerParams(
            needs_layout_passes=False, use_tc_tiling_on_sc=False),
    )(vals2d, pos3, luv2, cnt2)
    return out_pad[:BATCH], luo_pad[:BATCH]


def kernel(e_id, memory, last_update):
    e32 = e_id.astype(jnp.int32)
    mem_out, lu_out = _edge_gather(e32, memory.T, last_update)
    in_rem = e32 >= _REM_BASE
    e_rel = jnp.clip(e32 - _REM_BASE, 0, _REM_W - 1)
    mem_rem = jnp.take(memory[_REM_BASE:], e_rel, axis=0)
    lu_rem = jnp.take(last_update[_REM_BASE:], e_rel, axis=0)
    mem_out = jnp.where(in_rem[:, None], mem_rem, mem_out)
    lu_out = jnp.where(in_rem, lu_rem, lu_out)
    return (mem_out, lu_out.astype(last_update.dtype))


# speedup vs baseline: 22.3076x; 1.3316x over previous
import jax
import jax.numpy as jnp
from jax import lax
from jax.experimental import pallas as pl
from jax.experimental.pallas import tpu as pltpu
from jax.experimental.pallas import tpu_sc as plsc

NUM_EDGES = 1000000
MEMORY_DIM = 64
BATCH = 16384

_info = plsc.get_sparse_core_info()
_NC, _NS = _info.num_cores, _info.num_subcores
_NW = _NC * _NS
_CW = 512
_GPW = 61
_SPAN = _GPW * _CW
_EXTRA_BASE = _NW * _SPAN
_REM_BASE = _EXTRA_BASE + _CW
_REM_W = NUM_EDGES - _REM_BASE
_CAPW = 2048
_CAPC = 128
_NB = BATCH + _NW * _CAPW


def _scan_body(idx_hbm, memt_hbm, lu_hbm,
               vals_hbm, pos_hbm, luv_hbm, cnt_hbm,
               idx_v, me_v, mi_v, ce_v, ci_v,
               chunk_v, luch_v, stage_v, pos_acc, lu_acc,
               csem0, csem1, fsem):
    wid = lax.axis_index("s") * _NC + lax.axis_index("c")
    lo = wid * _SPAN
    hi = jnp.where(wid == _NW - 1, _REM_BASE, lo + _SPAN)

    pltpu.sync_copy(idx_hbm, idx_v)

    def f1(it, m):
        v = idx_v[pl.ds(it * 16, 16)]
        p = lax.iota(jnp.int32, 16) + it * 16
        mask = (v >= lo) & (v < hi)
        m_use = jnp.minimum(m, _CAPW - 16)
        plsc.store_compressed(me_v.at[pl.ds(m_use, 16)], v, mask=mask)
        plsc.store_compressed(mi_v.at[pl.ds(m_use, 16)], p, mask=mask)
        return m + plsc.all_reduce_population_count(mask)[0]

    m = lax.fori_loop(0, BATCH // 16, f1, jnp.int32(0))
    me_v[pl.ds(jnp.minimum(m, _CAPW - 16), 16)] = jnp.full((16,), -1, jnp.int32)

    dummy0 = BATCH + wid * _CAPW

    def pf(i, _):
        pos_acc[pl.ds(i * 16, 16)] = lax.iota(jnp.int32, 16) + (
            dummy0 + i * 16)
        return _

    lax.fori_loop(0, _CAPW // 16, pf, jnp.int32(0))

    n_groups = (m + 15) // 16

    def process(base, width, parity, total_pad):
        def f2(it, m2):
            v = me_v[pl.ds(it * 16, 16)]
            p = mi_v[pl.ds(it * 16, 16)]
            mask = (v >= base) & (v < base + width)
            m2_use = jnp.minimum(m2, _CAPC - 16)
            plsc.store_compressed(ce_v.at[pl.ds(m2_use, 16)], v - base,
                                  mask=mask)
            plsc.store_compressed(ci_v.at[pl.ds(m2_use, 16)], p, mask=mask)
            return m2 + plsc.all_reduce_population_count(mask)[0]

        m2 = lax.fori_loop(0, n_groups, f2, jnp.int32(0))
        m2 = jnp.minimum(m2, _CAPC - 16)
        ce_v[pl.ds(m2, 16)] = jnp.zeros((16,), jnp.int32)
        ci_v[pl.ds(m2, 16)] = lax.iota(jnp.int32, 16) + (
            dummy0 + total_pad + m2)
        n_g2 = (m2 + 15) // 16
        pv = jnp.full((16,), parity, jnp.int32)
        lubase = parity * _CW

        def extract(g2, _):
            cols16 = ce_v[pl.ds(g2 * 16, 16)]
            pos16 = ci_v[pl.ds(g2 * 16, 16)]
            lu16 = plsc.load_gather(luch_v, [lubase + cols16])
            off = jnp.minimum(total_pad + g2 * 16, _CAPW - 16)
            pos_acc[pl.ds(off, 16)] = pos16
            lu_acc[pl.ds(off, 16)] = lu16
            sbase = (parity * _CAPC + g2 * 16) * MEMORY_DIM
            for k in range(16):
                colv = jnp.full((16,), cols16[k], jnp.int32)
                for j in range(MEMORY_DIM // 16):
                    rows = lax.iota(jnp.int32, 16) + j * 16
                    vals = plsc.load_gather(chunk_v, [pv, rows, colv])
                    stage_v[pl.ds(sbase + k * MEMORY_DIM + j * 16, 16)] = vals
            return _

        lax.fori_loop(0, n_g2, extract, jnp.int32(0))

        def flush(u, _):
            row = jnp.minimum(total_pad + u * 16, _CAPW - 16)
            doff = pl.multiple_of((wid * _CAPW + row) * MEMORY_DIM, 1024)
            soff = pl.multiple_of(
                (parity * _CAPC + u * 16) * MEMORY_DIM, 1024)
            pltpu.async_copy(stage_v.at[pl.ds(soff, 16 * MEMORY_DIM)],
                             vals_hbm.at[pl.ds(doff, 16 * MEMORY_DIM)], fsem)
            return _

        lax.fori_loop(0, n_g2, flush, jnp.int32(0))
        return total_pad + n_g2 * 16, n_g2

    chunk_like = memt_hbm.at[:, pl.ds(0, _CW)]
    lu_like = lu_hbm.at[pl.ds(0, _CW)]

    def start_load(g, parity_slot, sem):
        base = pl.multiple_of(lo + g * _CW, _CW)
        pltpu.async_copy(memt_hbm.at[:, pl.ds(base, _CW)],
                         chunk_v.at[parity_slot], sem)
        pltpu.async_copy(lu_hbm.at[pl.ds(base, _CW)],
                         luch_v.at[pl.ds(parity_slot * _CW, _CW)], sem)

    start_load(jnp.int32(0), 0, csem0)

    def drain_unit(i, _):
        pltpu.make_async_copy(
            vals_hbm.at[pl.ds(0, 16 * MEMORY_DIM)],
            stage_v.at[pl.ds(0, 16 * MEMORY_DIM)], fsem).wait()
        return _

    def body(g, carry):
        total_pad, u0, u1 = carry
        parity = g % 2

        @pl.when(parity == 0)
        def _():
            pltpu.make_async_copy(chunk_like, chunk_v.at[0], csem0).wait()
            pltpu.make_async_copy(
                lu_like, luch_v.at[pl.ds(0, _CW)], csem0).wait()

        @pl.when(parity == 1)
        def _():
            pltpu.make_async_copy(chunk_like, chunk_v.at[1], csem1).wait()
            pltpu.make_async_copy(
                lu_like, luch_v.at[pl.ds(_CW, _CW)], csem1).wait()

        @pl.when((g + 1 < _GPW) & (parity == 0))
        def _():
            start_load(g + 1, 1, csem1)

        @pl.when((g + 1 < _GPW) & (parity == 1))
        def _():
            start_load(g + 1, 0, csem0)

        u_prev = jnp.where(parity == 0, u0, u1)
        lax.fori_loop(0, u_prev, drain_unit, jnp.int32(0))

        base = lo + g * _CW
        total_pad, n_u = process(base, _CW, parity, total_pad)
        u0 = jnp.where(parity == 0, n_u, u0)
        u1 = jnp.where(parity == 1, n_u, u1)
        return total_pad, u0, u1

    total_pad, u0, u1 = lax.fori_loop(
        0, _GPW, body, (jnp.int32(0), jnp.int32(0), jnp.int32(0)))

    lax.fori_loop(0, u0 + u1, drain_unit, jnp.int32(0))

    def write_count(tp):
        ce_v[pl.ds(0, 16)] = jnp.full((16,), tp, jnp.int32)
        coff = pl.multiple_of(wid * 16, 16)
        pltpu.sync_copy(ce_v.at[pl.ds(0, 16)], cnt_hbm.at[pl.ds(coff, 16)])

    @pl.when(wid == _NW - 1)
    def _():
        pltpu.sync_copy(memt_hbm.at[:, pl.ds(_EXTRA_BASE, _CW)],
                        chunk_v.at[0])
        pltpu.sync_copy(lu_hbm.at[pl.ds(_EXTRA_BASE, _CW)],
                        luch_v.at[pl.ds(0, _CW)])
        tp2, nu2 = process(jnp.int32(_EXTRA_BASE), _CW, 0, total_pad)
        lax.fori_loop(0, nu2, drain_unit, jnp.int32(0))
        write_count(tp2)

    @pl.when(wid != _NW - 1)
    def _():
        write_count(total_pad)

    poff = pl.multiple_of(wid * _CAPW, _CAPW)
    pltpu.sync_copy(pos_acc, pos_hbm.at[pl.ds(poff, _CAPW)])
    pltpu.sync_copy(lu_acc, luv_hbm.at[pl.ds(poff, _CAPW)])


def _scatter_body(vals_hbm, pos3_hbm, luv_hbm, cnt_hbm, out_hbm, luo_hbm,
                  pidx_v, luv_v, cnt_v, vstage_v, ssem0, ssem1, wsem):
    wid = lax.axis_index("s") * _NC + lax.axis_index("c")
    pltpu.sync_copy(pos3_hbm.at[wid], pidx_v)
    pltpu.sync_copy(luv_hbm.at[wid], luv_v)
    pltpu.sync_copy(cnt_hbm.at[wid], cnt_v)
    cnt = cnt_v[pl.ds(0, 16)][0]
    nj = (cnt + 127) // 128

    def stage(j, slot, sem):
        pltpu.async_copy(
            vals_hbm.at[pl.ds(wid * _CAPW + j * 128, 128)],
            vstage_v.at[slot], sem)

    def drain_stage0(i, _):
        pltpu.make_async_copy(vals_hbm.at[pl.ds(0, 128)],
                              vstage_v.at[0], ssem0).wait()
        return _

    def drain_stage1(i, _):
        pltpu.make_async_copy(vals_hbm.at[pl.ds(0, 128)],
                              vstage_v.at[1], ssem1).wait()
        return _

    def drain_scat(i, _):
        pltpu.make_async_copy(vstage_v.at[0],
                              out_hbm.at[pidx_v.at[0]], wsem).wait()
        pltpu.make_async_copy(luv_v.at[pl.ds(0, 128)],
                              luo_hbm.at[pidx_v.at[0]], wsem).wait()
        return _

    stage(jnp.int32(0), 0, ssem0)

    def body(j, carry):
        parity = j % 2
        lax.fori_loop(0, jnp.where(j >= 1, 1, 0), drain_scat, jnp.int32(0))

        @pl.when(parity == 0)
        def _():
            stage(j + 1, 1, ssem1)
            lax.fori_loop(0, 1, drain_stage0, jnp.int32(0))
            pltpu.async_copy(vstage_v.at[0], out_hbm.at[pidx_v.at[j]], wsem)

        @pl.when(parity == 1)
        def _():
            stage(j + 1, 0, ssem0)
            lax.fori_loop(0, 1, drain_stage1, jnp.int32(0))
            pltpu.async_copy(vstage_v.at[1], out_hbm.at[pidx_v.at[j]], wsem)

        pltpu.async_copy(luv_v.at[pl.ds(pl.multiple_of(j * 128, 128), 128)],
                         luo_hbm.at[pidx_v.at[j]], wsem)
        return carry

    lax.fori_loop(0, nj, body, jnp.int32(0))
    lax.fori_loop(0, jnp.minimum(nj, 1), drain_scat, jnp.int32(0))

    @pl.when(nj % 2 == 0)
    def _():
        lax.fori_loop(0, 1, drain_stage0, jnp.int32(0))

    @pl.when(nj % 2 == 1)
    def _():
        lax.fori_loop(0, 1, drain_stage1, jnp.int32(0))


@jax.jit
def _edge_gather(e_id32, memt, last_update):
    mesh = plsc.VectorSubcoreMesh(core_axis_name="c", subcore_axis_name="s")
    vals, pos, luv, cnt = pl.kernel(
        _scan_body,
        mesh=mesh,
        out_type=(
            jax.ShapeDtypeStruct(((_NW * _CAPW + 128) * MEMORY_DIM,),
                                 jnp.float32),
            jax.ShapeDtypeStruct((_NW * _CAPW,), jnp.int32),
            jax.ShapeDtypeStruct((_NW * _CAPW,), jnp.int32),
            jax.ShapeDtypeStruct((_NW * 16,), jnp.int32),
        ),
        scratch_types=[
            pltpu.VMEM((BATCH,), jnp.int32),
            pltpu.VMEM((_CAPW,), jnp.int32),
            pltpu.VMEM((_CAPW,), jnp.int32),
            pltpu.VMEM((_CAPC,), jnp.int32),
            pltpu.VMEM((_CAPC,), jnp.int32),
            pltpu.VMEM((2, MEMORY_DIM, _CW), jnp.float32),
            pltpu.VMEM((2 * _CW,), jnp.int32),
            pltpu.VMEM((2 * _CAPC * MEMORY_DIM,), jnp.float32),
            pltpu.VMEM((_CAPW,), jnp.int32),
            pltpu.VMEM((_CAPW,), jnp.int32),
            pltpu.SemaphoreType.DMA,
            pltpu.SemaphoreType.DMA,
            pltpu.SemaphoreType.DMA,
        ],
        compiler_params=pltpu.CompilerParams(needs_layout_passes=False),
    )(e_id32, memt, last_update)

    vals2d = vals.reshape(_NW * _CAPW + 128, MEMORY_DIM)
    pos3 = pos.reshape(_NW, _CAPW // 128, 128)
    luv2 = luv.reshape(_NW, _CAPW)
    cnt2 = cnt.reshape(_NW, 16)
    out_pad, luo_pad = pl.kernel(
        _scatter_body,
        mesh=mesh,
        out_type=(
            jax.ShapeDtypeStruct((_NB, MEMORY_DIM), jnp.float32),
            jax.ShapeDtypeStruct((_NB,), jnp.int32),
        ),
        scratch_types=[
            pltpu.VMEM((_CAPW // 128, 128), jnp.int32),
            pltpu.VMEM((_CAPW,), jnp.int32),
            pltpu.VMEM((16,), jnp.int32),
            pltpu.VMEM((2, 128, MEMORY_DIM), jnp.float32),
            pltpu.SemaphoreType.DMA,
            pltpu.SemaphoreType.DMA,
            pltpu.SemaphoreType.DMA,
        ],
        compiler_params=pltpu.CompilerParams(
            needs_layout_passes=False, use_tc_tiling_on_sc=False),
    )(vals2d, pos3, luv2, cnt2)
    return out_pad[:BATCH], luo_pad[:BATCH]


def kernel(e_id, memory, last_update):
    e32 = e_id.astype(jnp.int32)
    mem_out, lu_out = _edge_gather(e32, memory.T, last_update)
    in_rem = e32 >= _REM_BASE
    e_rel = jnp.clip(e32 - _REM_BASE, 0, _REM_W - 1)
    mem_rem = jnp.take(memory[_REM_BASE:], e_rel, axis=0)
    lu_rem = jnp.take(last_update[_REM_BASE:], e_rel, axis=0)
    mem_out = jnp.where(in_rem[:, None], mem_rem, mem_out)
    lu_out = jnp.where(in_rem, lu_rem, lu_out)
    return (mem_out, lu_out.astype(last_update.dtype))


# speedup vs baseline: 32.5986x; 1.4613x over previous
import jax
import jax.numpy as jnp
from jax import lax
from jax.experimental import pallas as pl
from jax.experimental.pallas import tpu as pltpu
from jax.experimental.pallas import tpu_sc as plsc

NUM_EDGES = 1000000
MEMORY_DIM = 64
BATCH = 16384

_info = plsc.get_sparse_core_info()
_NC, _NS = _info.num_cores, _info.num_subcores
_NW = _NC * _NS
_CW = 512
_GPW = 61
_SPAN = _GPW * _CW
_EXTRA_BASE = _NW * _SPAN
_REM_BASE = _EXTRA_BASE + _CW
_REM_W = NUM_EDGES - _REM_BASE
_CAPW = 2048
_CAPC = 128
_NB = BATCH + _NW * _CAPW


def _scan_body(idx_hbm, memt_hbm, lu_hbm, out_hbm, luo_hbm,
               idx_v, me_v, mi_v, ce_v, ci_v,
               chunk_v, luch_v, stage_v, lust_v,
               csem0, csem1, fsem):
    wid = lax.axis_index("s") * _NC + lax.axis_index("c")
    lo = wid * _SPAN
    hi = jnp.where(wid == _NW - 1, _REM_BASE, lo + _SPAN)
    dummy0 = BATCH + wid * _CAPW

    pltpu.sync_copy(idx_hbm, idx_v)

    def f1(it, m):
        v = idx_v[pl.ds(it * 16, 16)]
        p = lax.iota(jnp.int32, 16) + it * 16
        mask = (v >= lo) & (v < hi)
        m_use = jnp.minimum(m, _CAPW - 16)
        plsc.store_compressed(me_v.at[pl.ds(m_use, 16)], v, mask=mask)
        plsc.store_compressed(mi_v.at[pl.ds(m_use, 16)], p, mask=mask)
        return m + plsc.all_reduce_population_count(mask)[0]

    m = lax.fori_loop(0, BATCH // 16, f1, jnp.int32(0))
    me_v[pl.ds(jnp.minimum(m, _CAPW - 16), 16)] = jnp.full((16,), -1, jnp.int32)

    n_groups = (m + 15) // 16

    def process(base, parity, total_pad):
        def f2(it, m2):
            v = me_v[pl.ds(it * 16, 16)]
            p = mi_v[pl.ds(it * 16, 16)]
            mask = (v >= base) & (v < base + _CW)
            m2_use = jnp.minimum(m2, _CAPC - 16)
            plsc.store_compressed(ce_v.at[pl.ds(m2_use, 16)], v - base,
                                  mask=mask)
            plsc.store_compressed(ci_v.at[pl.ds(m2_use, 16)], p, mask=mask)
            return m2 + plsc.all_reduce_population_count(mask)[0]

        m2 = lax.fori_loop(0, n_groups, f2, jnp.int32(0))
        m2 = jnp.minimum(m2, _CAPC - 16)
        ce_v[pl.ds(m2, 16)] = jnp.zeros((16,), jnp.int32)
        ci_v[pl.ds(m2, 16)] = lax.iota(jnp.int32, 16) + (
            dummy0 + jnp.minimum(total_pad + m2, _CAPW - 16))
        n_g2 = (m2 + 15) // 16
        pv = jnp.full((16,), parity, jnp.int32)
        lubase = parity * _CW

        def extract(g2, _):
            cols16 = ce_v[pl.ds(g2 * 16, 16)]
            pos16 = ci_v[pl.ds(g2 * 16, 16)]
            lu16 = plsc.load_gather(luch_v, [lubase + cols16])
            sbase = (parity * _CAPC + g2 * 16) * MEMORY_DIM
            lbase = parity * _CAPC * 8 + g2 * 128
            plsc.store_scatter(lust_v,
                               [lbase + lax.iota(jnp.int32, 16) * 8], lu16)
            for k in range(16):
                colv = jnp.full((16,), cols16[k], jnp.int32)
                for j in range(MEMORY_DIM // 16):
                    rows = lax.iota(jnp.int32, 16) + j * 16
                    vals = plsc.load_gather(chunk_v, [pv, rows, colv])
                    stage_v[pl.ds(sbase + k * MEMORY_DIM + j * 16, 16)] = vals
                pos = pos16[k]
                pltpu.async_copy(
                    stage_v.at[pl.ds(sbase + k * MEMORY_DIM, MEMORY_DIM)],
                    out_hbm.at[pl.ds(
                        pl.multiple_of(pos * MEMORY_DIM, MEMORY_DIM),
                        MEMORY_DIM)], fsem)
                pltpu.async_copy(
                    lust_v.at[pl.ds(lbase + k * 8, 8)],
                    luo_hbm.at[pl.ds(pl.multiple_of(pos * 8, 8), 8)], fsem)
            return _

        lax.fori_loop(0, n_g2, extract, jnp.int32(0))
        return total_pad + n_g2 * 16, n_g2

    chunk_like = memt_hbm.at[:, pl.ds(0, _CW)]
    lu_like = lu_hbm.at[pl.ds(0, _CW)]

    def start_load(g, parity_slot, sem):
        base = pl.multiple_of(lo + g * _CW, _CW)
        pltpu.async_copy(memt_hbm.at[:, pl.ds(base, _CW)],
                         chunk_v.at[parity_slot], sem)
        pltpu.async_copy(lu_hbm.at[pl.ds(base, _CW)],
                         luch_v.at[pl.ds(parity_slot * _CW, _CW)], sem)

    start_load(jnp.int32(0), 0, csem0)

    def drain_unit(i, _):
        pltpu.make_async_copy(
            out_hbm.at[pl.ds(0, 16 * MEMORY_DIM)],
            stage_v.at[pl.ds(0, 16 * MEMORY_DIM)], fsem).wait()
        pltpu.make_async_copy(
            luo_hbm.at[pl.ds(0, 128)],
            lust_v.at[pl.ds(0, 128)], fsem).wait()
        return _

    def body(g, carry):
        total_pad, u0, u1 = carry
        parity = g % 2

        @pl.when(parity == 0)
        def _():
            pltpu.make_async_copy(chunk_like, chunk_v.at[0], csem0).wait()
            pltpu.make_async_copy(
                lu_like, luch_v.at[pl.ds(0, _CW)], csem0).wait()

        @pl.when(parity == 1)
        def _():
            pltpu.make_async_copy(chunk_like, chunk_v.at[1], csem1).wait()
            pltpu.make_async_copy(
                lu_like, luch_v.at[pl.ds(_CW, _CW)], csem1).wait()

        @pl.when((g + 1 < _GPW) & (parity == 0))
        def _():
            start_load(g + 1, 1, csem1)

        @pl.when((g + 1 < _GPW) & (parity == 1))
        def _():
            start_load(g + 1, 0, csem0)

        u_prev = jnp.where(parity == 0, u0, u1)
        lax.fori_loop(0, u_prev, drain_unit, jnp.int32(0))

        base = lo + g * _CW
        total_pad, n_u = process(base, parity, total_pad)
        u0 = jnp.where(parity == 0, n_u, u0)
        u1 = jnp.where(parity == 1, n_u, u1)
        return total_pad, u0, u1

    total_pad, u0, u1 = lax.fori_loop(
        0, _GPW, body, (jnp.int32(0), jnp.int32(0), jnp.int32(0)))

    lax.fori_loop(0, u0 + u1, drain_unit, jnp.int32(0))

    @pl.when(wid == _NW - 1)
    def _():
        pltpu.sync_copy(memt_hbm.at[:, pl.ds(_EXTRA_BASE, _CW)],
                        chunk_v.at[0])
        pltpu.sync_copy(lu_hbm.at[pl.ds(_EXTRA_BASE, _CW)],
                        luch_v.at[pl.ds(0, _CW)])
        tp2, nu2 = process(jnp.int32(_EXTRA_BASE), 0, total_pad)
        lax.fori_loop(0, nu2, drain_unit, jnp.int32(0))


@jax.jit
def _edge_gather(e_id32, memt, last_update):
    mesh = plsc.VectorSubcoreMesh(core_axis_name="c", subcore_axis_name="s")
    out_flat, luo_flat = pl.kernel(
        _scan_body,
        mesh=mesh,
        out_type=(
            jax.ShapeDtypeStruct((_NB * MEMORY_DIM,), jnp.float32),
            jax.ShapeDtypeStruct((_NB * 8,), jnp.int32),
        ),
        scratch_types=[
            pltpu.VMEM((BATCH,), jnp.int32),
            pltpu.VMEM((_CAPW,), jnp.int32),
            pltpu.VMEM((_CAPW,), jnp.int32),
            pltpu.VMEM((_CAPC,), jnp.int32),
            pltpu.VMEM((_CAPC,), jnp.int32),
            pltpu.VMEM((2, MEMORY_DIM, _CW), jnp.float32),
            pltpu.VMEM((2 * _CW,), jnp.int32),
            pltpu.VMEM((2 * _CAPC * MEMORY_DIM,), jnp.float32),
            pltpu.VMEM((2 * _CAPC * 8,), jnp.int32),
            pltpu.SemaphoreType.DMA,
            pltpu.SemaphoreType.DMA,
            pltpu.SemaphoreType.DMA,
        ],
        compiler_params=pltpu.CompilerParams(needs_layout_passes=False),
    )(e_id32, memt, last_update)
    mem_out = out_flat[:BATCH * MEMORY_DIM].reshape(BATCH, MEMORY_DIM)
    lu_out = luo_flat[:BATCH * 8].reshape(BATCH, 8)[:, 0]
    return mem_out, lu_out


def kernel(e_id, memory, last_update):
    e32 = e_id.astype(jnp.int32)
    mem_out, lu_out = _edge_gather(e32, memory.T, last_update)
    in_rem = e32 >= _REM_BASE
    e_rel = jnp.clip(e32 - _REM_BASE, 0, _REM_W - 1)
    mem_rem = jnp.take(memory[_REM_BASE:], e_rel, axis=0)
    lu_rem = jnp.take(last_update[_REM_BASE:], e_rel, axis=0)
    mem_out = jnp.where(in_rem[:, None], mem_rem, mem_out)
    lu_out = jnp.where(in_rem, lu_rem, lu_out)
    return (mem_out, lu_out.astype(last_update.dtype))
